# baseline (speedup 1.0000x reference)
import functools

import jax
import jax.numpy as jnp
from jax import lax
from jax.experimental import pallas as pl
from jax.experimental.pallas import tpu as pltpu
from jax.experimental.pallas import tpu_sc as plsc

_MAX_POSITION = 2048


def _tc_phase_shift(tab_ref, out_ref, *, width):
    ncols = tab_ref.shape[1]
    for t in range(tab_ref.shape[0]):
        x = jnp.broadcast_to(tab_ref[t], (128, ncols))
        rolled = pltpu.roll(x, ncols - 127, axis=1, stride=1, stride_axis=0)
        out_ref[:, t, :] = rolled[:, :width]


def _sc_expand(
    phases_hbm, out_hbm, spmem, tilebuf, sem_stage, sem_out, sem_g, sem_s, *, length, depth
):
    c = lax.axis_index("c")
    s = lax.axis_index("s")
    n_waves = 128 // 4

    def stage(w):
        return pltpu.make_async_copy(
            phases_hbm.at[pl.ds(4 * w + 2 * c, 1)],
            spmem.at[lax.rem(w, 2)],
            sem_stage,
        )

    @pl.when(s == 0)
    def _():
        h = stage(0)
        h.start()
        h.wait()

    plsc.subcore_barrier()

    a = pl.multiple_of((length - 128) - 128 * s, 128)
    dq = depth // 8

    def wave(w, carry):
        nxt = w + 1
        prefetch = (s == 0) & (nxt < n_waves)

        @pl.when(prefetch)
        def _():
            stage(nxt).start()

        r1 = 4 * w + 2 * c
        rows = [
            pltpu.make_async_copy(
                spmem.at[lax.rem(w, 2), 0, :, pl.ds(a, length)],
                out_hbm.at[r1 + 128 * s],
                sem_out,
            )
        ]
        for h in rows:
            h.start()

        r2 = 4 * w + 2 * c + 1
        i2 = r2 + 128 * s

        def bounce(g):
            gather = pltpu.make_async_copy(
                phases_hbm.at[r2, pl.ds(g * dq, dq), pl.ds(a, length)],
                tilebuf.at[g % 3],
                sem_g,
            )
            scatter = pltpu.make_async_copy(
                tilebuf.at[g % 3],
                out_hbm.at[i2, pl.ds(g * dq, dq)],
                sem_s,
            )
            return gather, scatter

        nch = depth // dq
        chain = [bounce(g) for g in range(nch)]
        for k in range(3):
            chain[k][0].start()
        for k in range(nch):
            chain[k][0].wait()
            chain[k][1].start()
            chain[k][1].wait()
            if k + 3 < nch:
                chain[k + 3][0].start()

        for h in rows:
            h.wait()

        @pl.when(prefetch)
        def _():
            stage(nxt).wait()

        plsc.subcore_barrier()
        return carry

    lax.fori_loop(0, n_waves, wave, 0)


def kernel(inputs, rel_embeddings):
    length = inputs.shape[1]
    depth = rel_embeddings.shape[1]
    table_rows = rel_embeddings.shape[0]

    width = 2 * length - 128
    padded_cols = width + 128
    revt = rel_embeddings[::-1].T
    revt = jnp.pad(revt, ((0, 0), (0, padded_cols - table_rows)))

    phases = pl.pallas_call(
        functools.partial(_tc_phase_shift, width=width),
        grid=(depth // 8,),
        in_specs=[pl.BlockSpec((8, padded_cols), lambda d: (d, 0))],
        out_specs=pl.BlockSpec((128, 8, width), lambda d: (0, d, 0)),
        out_shape=jax.ShapeDtypeStruct((128, depth, width), jnp.float32),
    )(revt)

    mesh = plsc.VectorSubcoreMesh(core_axis_name="c", subcore_axis_name="s")
    body = functools.partial(_sc_expand, length=length, depth=depth)
    out = pl.kernel(
        body,
        mesh=mesh,
        out_type=jax.ShapeDtypeStruct((length, depth, length), jnp.float32),
        scratch_types=[
            pltpu.VMEM_SHARED((2, 1, depth, width), jnp.float32),
            pltpu.VMEM((3, depth // 8, length), jnp.float32),
            pltpu.SemaphoreType.DMA,
            pltpu.SemaphoreType.DMA,
            pltpu.SemaphoreType.DMA,
            pltpu.SemaphoreType.DMA,
        ],
    )(phases)
    return jnp.transpose(out, (0, 2, 1))


# speedup vs baseline: 1.2458x; 1.2458x over previous
import functools

import jax
import jax.numpy as jnp
from jax import lax
from jax.experimental import pallas as pl
from jax.experimental.pallas import tpu as pltpu
from jax.experimental.pallas import tpu_sc as plsc

_MAX_POSITION = 2048


def _tc_phase_shift(tab_ref, out_ref, *, width):
    ncols = tab_ref.shape[1]
    for t in range(tab_ref.shape[0]):
        x = jnp.broadcast_to(tab_ref[t], (128, ncols))
        rolled = pltpu.roll(x, ncols - 127, axis=1, stride=1, stride_axis=0)
        out_ref[:, t, :] = rolled[:, :width]


def _sc_expand(phases_hbm, out_hbm, spmem, sem_stage, sem_out, *, length, depth):
    c = lax.axis_index("c")
    s = lax.axis_index("s")
    ppw = 2
    n_waves = 128 // (2 * ppw)

    def stage(w):
        return pltpu.make_async_copy(
            phases_hbm.at[pl.ds(2 * ppw * w + ppw * c, ppw)],
            spmem.at[lax.rem(w, 2)],
            sem_stage,
        )

    @pl.when(s == 0)
    def _():
        h = stage(0)
        h.start()
        h.wait()

    plsc.subcore_barrier()

    def wave(w, carry):
        nxt = w + 1
        prefetch = (s == 0) & (nxt < n_waves)

        @pl.when(prefetch)
        def _():
            stage(nxt).start()

        a = pl.multiple_of(
            (length - 128) - 128 * s, 128
        )
        rows = []
        for q in range(ppw):
            r = 2 * ppw * w + ppw * c + q
            i = r + 128 * s
            rows.append(
                pltpu.make_async_copy(
                    spmem.at[lax.rem(w, 2), q, :, pl.ds(a, length)],
                    out_hbm.at[i],
                    sem_out,
                )
            )
        for h in rows:
            h.start()
        for h in rows:
            h.wait()

        @pl.when(prefetch)
        def _():
            stage(nxt).wait()

        plsc.subcore_barrier()
        return carry

    lax.fori_loop(0, n_waves, wave, 0)


def kernel(inputs, rel_embeddings):
    length = inputs.shape[1]
    depth = rel_embeddings.shape[1]
    table_rows = rel_embeddings.shape[0]

    width = 2 * length - 128
    padded_cols = width + 128
    revt = rel_embeddings[::-1].T
    revt = jnp.pad(revt, ((0, 0), (0, padded_cols - table_rows)))

    phases = pl.pallas_call(
        functools.partial(_tc_phase_shift, width=width),
        grid=(depth // 8,),
        in_specs=[pl.BlockSpec((8, padded_cols), lambda d: (d, 0))],
        out_specs=pl.BlockSpec((128, 8, width), lambda d: (0, d, 0)),
        out_shape=jax.ShapeDtypeStruct((128, depth, width), jnp.float32),
    )(revt)

    mesh = plsc.VectorSubcoreMesh(core_axis_name="c", subcore_axis_name="s")
    body = functools.partial(_sc_expand, length=length, depth=depth)
    out = pl.kernel(
        body,
        mesh=mesh,
        out_type=jax.ShapeDtypeStruct((length, depth, length), jnp.float32),
        scratch_types=[
            pltpu.VMEM_SHARED((2, 2, depth, width), jnp.float32),
            pltpu.SemaphoreType.DMA,
            pltpu.SemaphoreType.DMA,
        ],
    )(phases)
    return jnp.transpose(out, (0, 2, 1))
